# baseline (speedup 1.0000x reference)
import functools

import jax
import jax.numpy as jnp
from jax import lax
from jax.experimental import pallas as pl
from jax.experimental.pallas import tpu as pltpu
from jax.experimental.pallas import tpu_sc as plsc

_DIM = 2048
_N_EXPERTS = 64
_TOPK = 8
_ROWS = 8192
_BLOCK_R = 1024

_NC = 2
_NS = 16
_NW = _NC * _NS
_ROWS_PER_W = _ROWS // _NW
_GROUPS_PER_W = _ROWS_PER_W // 16


def _probs_block(x_ref, w_ref, p_ref):
    st = jax.lax.dot_general(
        w_ref[...], x_ref[...],
        (((1,), (1,)), ((), ())),
        preferred_element_type=jnp.float32,
    )
    m = jnp.max(st, axis=0, keepdims=True)
    e = jnp.exp(st - m)
    p_ref[...] = e * (1.0 / jnp.sum(e, axis=0, keepdims=True))


def _probs(x, weight):
    grid = (_ROWS // _BLOCK_R,)
    return pl.pallas_call(
        _probs_block,
        grid=grid,
        in_specs=[
            pl.BlockSpec((_BLOCK_R, _DIM), lambda i: (i, 0)),
            pl.BlockSpec((_N_EXPERTS, _DIM), lambda i: (0, 0)),
        ],
        out_specs=pl.BlockSpec((_N_EXPERTS, _BLOCK_R), lambda i: (0, i)),
        out_shape=jax.ShapeDtypeStruct((_N_EXPERTS, _ROWS), jnp.float32),
    )(x, weight)


def _topk_sc_body(p_hbm, wts_hbm, idx_hbm, p_v, wts_v, idx_v):
    wid = lax.axis_index("s") * _NC + lax.axis_index("c")
    base = wid * _ROWS_PER_W
    pltpu.sync_copy(p_hbm.at[:, pl.ds(base, _ROWS_PER_W)], p_v)

    for g in range(_GROUPS_PER_W):
        goff = g * 16

        def step(e, carry):
            ws, ids = carry
            v = p_v[e, pl.ds(goff, 16)]
            ie = jnp.full((16,), 1.0, jnp.float32) * e.astype(jnp.float32)
            cs = [v > w for w in ws]
            new_ws = list(ws)
            new_ids = list(ids)
            new_ws[0] = jnp.where(cs[0], v, ws[0])
            new_ids[0] = jnp.where(cs[0], ie, ids[0])
            for j in range(1, _TOPK):
                tw = jnp.where(cs[j - 1], ws[j - 1], v)
                ti = jnp.where(cs[j - 1], ids[j - 1], ie)
                new_ws[j] = jnp.where(cs[j], tw, ws[j])
                new_ids[j] = jnp.where(cs[j], ti, ids[j])
            return tuple(new_ws), tuple(new_ids)

        init_w = tuple(jnp.full((16,), -1.0, jnp.float32)
                       for _ in range(_TOPK))
        init_i = tuple(jnp.full((16,), 0.0, jnp.float32)
                       for _ in range(_TOPK))
        ws, ids = lax.fori_loop(0, _N_EXPERTS, step, (init_w, init_i))
        for j in range(_TOPK):
            wts_v[j, pl.ds(goff, 16)] = ws[j]
            idx_v[j, pl.ds(goff, 16)] = ids[j].astype(jnp.int32)

    pltpu.sync_copy(wts_v, wts_hbm.at[:, pl.ds(base, _ROWS_PER_W)])
    pltpu.sync_copy(idx_v, idx_hbm.at[:, pl.ds(base, _ROWS_PER_W)])


def _topk_sc(p):
    mesh = plsc.VectorSubcoreMesh(core_axis_name="c", subcore_axis_name="s")
    f = functools.partial(
        pl.kernel,
        mesh=mesh,
        out_type=[
            jax.ShapeDtypeStruct((_TOPK, _ROWS), jnp.float32),
            jax.ShapeDtypeStruct((_TOPK, _ROWS), jnp.int32),
        ],
        scratch_types=[
            pltpu.VMEM((_N_EXPERTS, _ROWS_PER_W), jnp.float32),
            pltpu.VMEM((_TOPK, _ROWS_PER_W), jnp.float32),
            pltpu.VMEM((_TOPK, _ROWS_PER_W), jnp.int32),
        ],
    )(_topk_sc_body)
    return f(p)


def kernel(x, weight):
    p = _probs(x, weight)
    wts_t, idx_t = _topk_sc(p)
    return wts_t.T, idx_t.T


# speedup vs baseline: 1.9831x; 1.9831x over previous
import jax
import jax.numpy as jnp
from jax.experimental import pallas as pl

_DIM = 2048
_N_EXPERTS = 64
_TOPK = 8
_ROWS = 8192
_BLOCK_R = 1024


def _gate_block(x_ref, w_ref, wts_ref, idx_ref):
    st = jax.lax.dot_general(
        w_ref[...], x_ref[...],
        (((1,), (1,)), ((), ())),
        preferred_element_type=jnp.float32,
    )
    m = jnp.max(st, axis=0, keepdims=True)
    e = jnp.exp(st - m)
    recip = 1.0 / jnp.sum(e, axis=0, keepdims=True)

    iota = jax.lax.broadcasted_iota(jnp.int32, st.shape, 0).astype(jnp.float32)
    wts_rows = []
    idx_rows = []
    work = st
    neg = jnp.float32(-jnp.inf)
    for k in range(_TOPK):
        mx = m if k == 0 else jnp.max(work, axis=0, keepdims=True)
        ix = jnp.min(jnp.where(work == mx, iota, jnp.float32(_N_EXPERTS)),
                     axis=0, keepdims=True)
        wts_rows.append(recip if k == 0 else jnp.exp(mx - m) * recip)
        idx_rows.append(ix)
        if k + 1 < _TOPK:
            work = jnp.where(iota == ix, neg, work)
    wts_ref[...] = jnp.concatenate(wts_rows, axis=0)
    idx_ref[...] = jnp.concatenate(idx_rows, axis=0).astype(jnp.int32)


def kernel(x, weight):
    grid = (_ROWS // _BLOCK_R,)
    wts_t, idx_t = pl.pallas_call(
        _gate_block,
        grid=grid,
        in_specs=[
            pl.BlockSpec((_BLOCK_R, _DIM), lambda i: (i, 0)),
            pl.BlockSpec((_N_EXPERTS, _DIM), lambda i: (0, 0)),
        ],
        out_specs=[
            pl.BlockSpec((_TOPK, _BLOCK_R), lambda i: (0, i)),
            pl.BlockSpec((_TOPK, _BLOCK_R), lambda i: (0, i)),
        ],
        out_shape=[
            jax.ShapeDtypeStruct((_TOPK, _ROWS), jnp.float32),
            jax.ShapeDtypeStruct((_TOPK, _ROWS), jnp.int32),
        ],
    )(x, weight)
    return wts_t.T, idx_t.T


# speedup vs baseline: 2.0032x; 1.0101x over previous
import jax
import jax.numpy as jnp
from jax.experimental import pallas as pl

_DIM = 2048
_N_EXPERTS = 64
_TOPK = 8
_ROWS = 8192
_BLOCK_R = 1024
_CHUNK = 256


def _gate_block(x_ref, w_ref, wts_ref, idx_ref):
    for c in range(_BLOCK_R // _CHUNK):
        sl = pl.ds(c * _CHUNK, _CHUNK)
        st = jax.lax.dot_general(
            w_ref[...], x_ref[sl, :],
            (((1,), (1,)), ((), ())),
            preferred_element_type=jnp.float32,
        )
        m = jnp.max(st, axis=0, keepdims=True)
        e = jnp.exp(st - m)
        recip = 1.0 / jnp.sum(e, axis=0, keepdims=True)

        iota = jax.lax.broadcasted_iota(
            jnp.int32, st.shape, 0).astype(jnp.float32)
        wts_rows = []
        idx_rows = []
        work = st
        neg = jnp.float32(-jnp.inf)
        for k in range(_TOPK):
            mx = m if k == 0 else jnp.max(work, axis=0, keepdims=True)
            ix = jnp.min(
                jnp.where(work == mx, iota, jnp.float32(_N_EXPERTS)),
                axis=0, keepdims=True)
            wts_rows.append(recip if k == 0 else jnp.exp(mx - m) * recip)
            idx_rows.append(ix)
            if k + 1 < _TOPK:
                work = jnp.where(iota == ix, neg, work)
        wts_ref[:, sl] = jnp.concatenate(wts_rows, axis=0)
        idx_ref[:, sl] = jnp.concatenate(idx_rows, axis=0).astype(jnp.int32)


def kernel(x, weight):
    grid = (_ROWS // _BLOCK_R,)
    wts_t, idx_t = pl.pallas_call(
        _gate_block,
        grid=grid,
        in_specs=[
            pl.BlockSpec((_BLOCK_R, _DIM), lambda i: (i, 0)),
            pl.BlockSpec((_N_EXPERTS, _DIM), lambda i: (0, 0)),
        ],
        out_specs=[
            pl.BlockSpec((_TOPK, _BLOCK_R), lambda i: (0, i)),
            pl.BlockSpec((_TOPK, _BLOCK_R), lambda i: (0, i)),
        ],
        out_shape=[
            jax.ShapeDtypeStruct((_TOPK, _ROWS), jnp.float32),
            jax.ShapeDtypeStruct((_TOPK, _ROWS), jnp.int32),
        ],
    )(x, weight)
    return wts_t.T, idx_t.T
